# Initial kernel scaffold; baseline (speedup 1.0000x reference)
import jax
import jax.numpy as jnp
from jax.experimental import pallas as pl


def kernel(x, edge_index, batch, W1, b1, W2, b2, W3, b3, Wfc1, bfc1, Wfc2, bfc2):
    raise NotImplementedError("write your pallas kernel here")


# speedup vs baseline: 14.0969x; 14.0969x over previous
import functools

import jax
import jax.numpy as jnp
from jax import lax
from jax.experimental import pallas as pl
from jax.experimental.pallas import tpu as pltpu
from jax.experimental.pallas import tpu_sc as plsc

N = 10000
E = 320000
NG = 500
NP = 10240
G = 512
DUMP = N

NC, NS, L = 2, 16, 16
NW = NC * NS
EC = 128
CH = 80
EPT = CH * EC
EPAD = NW * EPT

RB = 512
NB = NP // RB

_f32 = jnp.float32
_mesh = plsc.VectorSubcoreMesh(core_axis_name="c", subcore_axis_name="s")


DW = 8


@functools.partial(
    pl.kernel,
    out_type=jax.ShapeDtypeStruct((NC, NP, DW), _f32),
    mesh=_mesh,
    scratch_types=[
        pltpu.VMEM((CH, EC), jnp.int32),
        pltpu.VMEM((EC, DW), _f32),
        pltpu.VMEM_SHARED((NP, DW), _f32),
    ],
)
def _deg_kernel(dst_hbm, ones_hbm, zero_hbm, degp_hbm, dst_v, buf, acc):
    c = lax.axis_index("c")
    s = lax.axis_index("s")
    t = c * NS + s
    rpt = NP // NS
    pltpu.sync_copy(zero_hbm.at[pl.ds(s * rpt, rpt)],
                    acc.at[pl.ds(s * rpt, rpt)])
    pltpu.sync_copy(dst_hbm.at[t], dst_v)
    pltpu.sync_copy(ones_hbm, buf)
    plsc.subcore_barrier()

    def body(j, _):
        pltpu.sync_copy(buf, acc.at[dst_v.at[j]], add=True)
        return 0

    lax.fori_loop(0, CH, body, 0)
    plsc.subcore_barrier()
    pltpu.sync_copy(acc.at[pl.ds(s * rpt, rpt)],
                    degp_hbm.at[c, pl.ds(s * rpt, rpt)])


def _make_scatter(D, ec, nh):
    ch = EPT // ec
    chh = ch // nh
    rpt = NP // NS

    @functools.partial(
        pl.kernel,
        out_type=jax.ShapeDtypeStruct((NC, NP, D), _f32),
        mesh=_mesh,
        scratch_types=[
            pltpu.VMEM((chh, ec), jnp.int32),
            pltpu.VMEM((chh, ec), jnp.int32),
            pltpu.VMEM((2, ec, D), _f32),
            pltpu.VMEM_SHARED((NP, D), _f32),
            pltpu.SemaphoreType.DMA,
            pltpu.SemaphoreType.DMA,
        ],
        compiler_params=pltpu.CompilerParams(use_tc_tiling_on_sc=False),
    )
    def _scatter(zp_hbm, src_hbm, dst_hbm, zero_hbm, out_hbm,
                 src_v, dst_v, buf, acc, sem0, sem1):
        c = lax.axis_index("c")
        s = lax.axis_index("s")
        t = c * NS + s
        pltpu.sync_copy(zero_hbm.at[pl.ds(s * rpt, rpt)],
                        acc.at[pl.ds(s * rpt, rpt)])
        plsc.subcore_barrier()

        for h in range(nh):
            pltpu.sync_copy(src_hbm.at[t, pl.ds(h * chh, chh)], src_v)
            pltpu.sync_copy(dst_hbm.at[t, pl.ds(h * chh, chh)], dst_v)

            pltpu.async_copy(zp_hbm.at[src_v.at[0]], buf.at[0], sem0)
            pltpu.async_copy(zp_hbm.at[src_v.at[1]], buf.at[1], sem1)

            def outer(g, _):
                for b, sem in ((0, sem0), (1, sem1)):
                    j = 2 * g + b
                    pltpu.make_async_copy(zp_hbm.at[src_v.at[j]],
                                          buf.at[b], sem).wait()
                    pltpu.sync_copy(buf.at[b], acc.at[dst_v.at[j]], add=True)

                    @pl.when(j + 2 < chh)
                    def _refill():
                        pltpu.async_copy(zp_hbm.at[src_v.at[j + 2]],
                                         buf.at[b], sem)

                return 0

            lax.fori_loop(0, chh // 2, outer, 0)

        plsc.subcore_barrier()
        pltpu.sync_copy(acc.at[pl.ds(s * rpt, rpt)],
                        out_hbm.at[c, pl.ds(s * rpt, rpt)])

    return _scatter


EC128 = 64
_scatter128 = _make_scatter(128, EC128, 2)
_scatter64 = _make_scatter(64, EC, 1)
_scatter32 = _make_scatter(32, EC, 1)


def _dinv_of(degp_blk):
    return lax.rsqrt(1.0 + degp_blk[0, :, 0] + degp_blk[1, :, 0])


def _layer1_body(x_ref, w_ref, degp_ref, out_ref):
    dinv = _dinv_of(degp_ref[...])
    z = jnp.dot(x_ref[...], w_ref[...], preferred_element_type=_f32)
    out_ref[...] = dinv[:, None] * z


def _layer_body(p_ref, zp_ref, degp_ref, b_ref, w_ref, out_ref):
    dinv = _dinv_of(degp_ref[...])
    sm = p_ref[0] + p_ref[1] + zp_ref[...]
    h = jnp.maximum(dinv[:, None] * sm + b_ref[...], 0.0)
    z = jnp.dot(h, w_ref[...], preferred_element_type=_f32)
    out_ref[...] = dinv[:, None] * z


def _tc_layer1(xp, W1, degp):
    return pl.pallas_call(
        _layer1_body,
        grid=(NB,),
        in_specs=[
            pl.BlockSpec((RB, 128), lambda i: (i, 0)),
            pl.BlockSpec((128, 128), lambda i: (0, 0)),
            pl.BlockSpec((NC, RB, DW), lambda i: (0, i, 0)),
        ],
        out_specs=pl.BlockSpec((RB, 128), lambda i: (i, 0)),
        out_shape=jax.ShapeDtypeStruct((NP, 128), _f32),
    )(xp, W1, degp)


def _tc_layer(p, zp, degp, b2d, W):
    di, do = W.shape
    return pl.pallas_call(
        _layer_body,
        grid=(NB,),
        in_specs=[
            pl.BlockSpec((NC, RB, di), lambda i: (0, i, 0)),
            pl.BlockSpec((RB, di), lambda i: (i, 0)),
            pl.BlockSpec((NC, RB, DW), lambda i: (0, i, 0)),
            pl.BlockSpec((1, di), lambda i: (0, 0)),
            pl.BlockSpec((di, do), lambda i: (0, 0)),
        ],
        out_specs=pl.BlockSpec((RB, do), lambda i: (i, 0)),
        out_shape=jax.ShapeDtypeStruct((NP, do), _f32),
    )(p, zp, degp, b2d, W)


def _pool_body(p_ref, zp_ref, degp_ref, b3_ref, batch_ref,
               wf1_ref, bf1_ref, wf2_ref, bf2_ref, out_ref,
               sums_ref, cnts_ref):
    i = pl.program_id(0)

    @pl.when(i == 0)
    def _init():
        sums_ref[...] = jnp.zeros_like(sums_ref)
        cnts_ref[...] = jnp.zeros_like(cnts_ref)

    dinv = _dinv_of(degp_ref[...])
    sm = p_ref[0] + p_ref[1] + zp_ref[...]
    h3 = jnp.maximum(dinv[:, None] * sm + b3_ref[...], 0.0)
    b = batch_ref[0]
    gids = lax.broadcasted_iota(jnp.int32, (G, RB), 0)
    onehot = (gids == b).astype(_f32)
    sums_ref[...] += jnp.dot(onehot, h3, preferred_element_type=_f32)
    cnts_ref[...] += jnp.sum(onehot, axis=1, keepdims=True)

    @pl.when(i == NB - 1)
    def _final():
        g = sums_ref[...] / jnp.maximum(cnts_ref[...], 1.0)
        f = jnp.maximum(jnp.dot(g, wf1_ref[...],
                                preferred_element_type=_f32) + bf1_ref[...], 0.0)
        out_ref[...] = jnp.dot(f, wf2_ref[...],
                               preferred_element_type=_f32) + bf2_ref[...]


def _tc_pool(p, zp, degp, b3_2d, batch3, Wfc1, bfc1_2d, Wfc2, bfc2_2d):
    return pl.pallas_call(
        _pool_body,
        grid=(NB,),
        in_specs=[
            pl.BlockSpec((NC, RB, 32), lambda i: (0, i, 0)),
            pl.BlockSpec((RB, 32), lambda i: (i, 0)),
            pl.BlockSpec((NC, RB, DW), lambda i: (0, i, 0)),
            pl.BlockSpec((1, 32), lambda i: (0, 0)),
            pl.BlockSpec((1, 1, RB), lambda i: (i, 0, 0)),
            pl.BlockSpec((32, 16), lambda i: (0, 0)),
            pl.BlockSpec((1, 16), lambda i: (0, 0)),
            pl.BlockSpec((16, 1), lambda i: (0, 0)),
            pl.BlockSpec((1, 1), lambda i: (0, 0)),
        ],
        out_specs=pl.BlockSpec((G, 1), lambda i: (0, 0)),
        out_shape=jax.ShapeDtypeStruct((G, 1), _f32),
        scratch_shapes=[
            pltpu.VMEM((G, 32), _f32),
            pltpu.VMEM((G, 1), _f32),
        ],
    )(p, zp, degp, b3_2d, batch3, Wfc1, bfc1_2d, Wfc2, bfc2_2d)


def kernel(x, edge_index, batch, W1, b1, W2, b2, W3, b3,
           Wfc1, bfc1, Wfc2, bfc2):
    src_flat = jnp.concatenate(
        [edge_index[0], jnp.zeros((EPAD - E,), jnp.int32)])
    dst_flat = jnp.concatenate(
        [edge_index[1], jnp.full((EPAD - E,), DUMP, jnp.int32)])
    src = src_flat.reshape(NW, CH, EC)
    dst = dst_flat.reshape(NW, CH, EC)
    src_n = src_flat.reshape(NW, EPT // EC128, EC128)
    dst_n = dst_flat.reshape(NW, EPT // EC128, EC128)

    xp = jnp.pad(x, ((0, NP - N), (0, 0)))
    batch3 = jnp.pad(batch, (0, NP - N), constant_values=G - 1).reshape(NB, 1, RB)

    degp = _deg_kernel(dst, jnp.ones((EC, DW), _f32), jnp.zeros((NP, DW), _f32))

    z1p = _tc_layer1(xp, W1, degp)
    p1 = _scatter128(z1p, src_n, dst_n, jnp.zeros((NP, 128), _f32))
    z2p = _tc_layer(p1, z1p, degp, b1.reshape(1, -1), W2)
    p2 = _scatter64(z2p, src, dst, jnp.zeros((NP, 64), _f32))
    z3p = _tc_layer(p2, z2p, degp, b2.reshape(1, -1), W3)
    p3 = _scatter32(z3p, src, dst, jnp.zeros((NP, 32), _f32))

    out = _tc_pool(p3, z3p, degp, b3.reshape(1, -1), batch3,
                   Wfc1, bfc1.reshape(1, -1), Wfc2, bfc2.reshape(1, -1))
    return out[:NG]


# speedup vs baseline: 14.3692x; 1.0193x over previous
import functools

import jax
import jax.numpy as jnp
from jax import lax
from jax.experimental import pallas as pl
from jax.experimental.pallas import tpu as pltpu
from jax.experimental.pallas import tpu_sc as plsc

N = 10000
E = 320000
NG = 500
NP = 10240
G = 512
DUMP = N

NC, NS, L = 2, 16, 16
NW = NC * NS
EC = 128
CH = 80
EPT = CH * EC
EPAD = NW * EPT

RB = 512
NB = NP // RB

_f32 = jnp.float32
_mesh = plsc.VectorSubcoreMesh(core_axis_name="c", subcore_axis_name="s")


DW = 8


@functools.partial(
    pl.kernel,
    out_type=jax.ShapeDtypeStruct((NC, NP, DW), _f32),
    mesh=_mesh,
    scratch_types=[
        pltpu.VMEM((CH, EC), jnp.int32),
        pltpu.VMEM((EC, DW), _f32),
        pltpu.VMEM_SHARED((NP, DW), _f32),
    ],
)
def _deg_kernel(dst_hbm, ones_hbm, zero_hbm, degp_hbm, dst_v, buf, acc):
    c = lax.axis_index("c")
    s = lax.axis_index("s")
    t = c * NS + s
    rpt = NP // NS
    pltpu.sync_copy(zero_hbm.at[pl.ds(s * rpt, rpt)],
                    acc.at[pl.ds(s * rpt, rpt)])
    pltpu.sync_copy(dst_hbm.at[t], dst_v)
    pltpu.sync_copy(ones_hbm, buf)
    plsc.subcore_barrier()

    def body(j, _):
        pltpu.sync_copy(buf, acc.at[dst_v.at[j]], add=True)
        return 0

    lax.fori_loop(0, CH, body, 0)
    plsc.subcore_barrier()
    pltpu.sync_copy(acc.at[pl.ds(s * rpt, rpt)],
                    degp_hbm.at[c, pl.ds(s * rpt, rpt)])


EPP = 2560
PIECES = EPAD // EPP
N_FAST = 6
N_SLOW = 8 - N_FAST
FC = 0


def _make_scatter(D, ec):
    P = EPP // ec
    rpt = NP // NS

    @functools.partial(
        pl.kernel,
        out_type=jax.ShapeDtypeStruct((NC, NP, D), _f32),
        mesh=_mesh,
        scratch_types=[
            pltpu.VMEM((P, ec), jnp.int32),
            pltpu.VMEM((P, ec), jnp.int32),
            pltpu.VMEM((2, ec, D), _f32),
            pltpu.VMEM_SHARED((NP, D), _f32),
            pltpu.SemaphoreType.DMA,
            pltpu.SemaphoreType.DMA,
        ],
        compiler_params=pltpu.CompilerParams(use_tc_tiling_on_sc=False),
    )
    def _scatter(zp_hbm, src_hbm, dst_hbm, zero_hbm, out_hbm,
                 src_v, dst_v, buf, acc, sem0, sem1):
        c = lax.axis_index("c")
        s = lax.axis_index("s")
        pltpu.sync_copy(zero_hbm.at[pl.ds(s * rpt, rpt)],
                        acc.at[pl.ds(s * rpt, rpt)])
        plsc.subcore_barrier()

        is_fast = c == FC
        n_pieces = jnp.where(is_fast, N_FAST, N_SLOW)
        base = jnp.where(is_fast, s * N_FAST, NS * N_FAST + s * N_SLOW)

        def run_piece(p, _):
            pc = base + p
            pltpu.sync_copy(src_hbm.at[pl.ds(pc * P, P)], src_v)
            pltpu.sync_copy(dst_hbm.at[pl.ds(pc * P, P)], dst_v)

            pltpu.async_copy(zp_hbm.at[src_v.at[0]], buf.at[0], sem0)
            pltpu.async_copy(zp_hbm.at[src_v.at[1]], buf.at[1], sem1)

            def outer(g, _):
                for b, sem in ((0, sem0), (1, sem1)):
                    j = 2 * g + b
                    pltpu.make_async_copy(zp_hbm.at[src_v.at[j]],
                                          buf.at[b], sem).wait()
                    pltpu.sync_copy(buf.at[b], acc.at[dst_v.at[j]], add=True)

                    @pl.when(j + 2 < P)
                    def _refill():
                        pltpu.async_copy(zp_hbm.at[src_v.at[j + 2]],
                                         buf.at[b], sem)

                return 0

            lax.fori_loop(0, P // 2, outer, 0)
            return 0

        lax.fori_loop(0, n_pieces, run_piece, 0)

        plsc.subcore_barrier()
        pltpu.sync_copy(acc.at[pl.ds(s * rpt, rpt)],
                        out_hbm.at[c, pl.ds(s * rpt, rpt)])

    return _scatter


EC128 = 64
_scatter128 = _make_scatter(128, EC128)
_scatter64 = _make_scatter(64, EC)
_scatter32 = _make_scatter(32, EC)


def _dinv_of(degp_blk):
    return lax.rsqrt(1.0 + degp_blk[0, :, 0] + degp_blk[1, :, 0])


def _layer1_body(x_ref, w_ref, degp_ref, out_ref):
    dinv = _dinv_of(degp_ref[...])
    z = jnp.dot(x_ref[...], w_ref[...], preferred_element_type=_f32)
    out_ref[...] = dinv[:, None] * z


def _layer_body(p_ref, zp_ref, degp_ref, b_ref, w_ref, out_ref):
    dinv = _dinv_of(degp_ref[...])
    sm = p_ref[0] + p_ref[1] + zp_ref[...]
    h = jnp.maximum(dinv[:, None] * sm + b_ref[...], 0.0)
    z = jnp.dot(h, w_ref[...], preferred_element_type=_f32)
    out_ref[...] = dinv[:, None] * z


def _tc_layer1(xp, W1, degp):
    return pl.pallas_call(
        _layer1_body,
        grid=(NB,),
        in_specs=[
            pl.BlockSpec((RB, 128), lambda i: (i, 0)),
            pl.BlockSpec((128, 128), lambda i: (0, 0)),
            pl.BlockSpec((NC, RB, DW), lambda i: (0, i, 0)),
        ],
        out_specs=pl.BlockSpec((RB, 128), lambda i: (i, 0)),
        out_shape=jax.ShapeDtypeStruct((NP, 128), _f32),
    )(xp, W1, degp)


def _tc_layer(p, zp, degp, b2d, W):
    di, do = W.shape
    return pl.pallas_call(
        _layer_body,
        grid=(NB,),
        in_specs=[
            pl.BlockSpec((NC, RB, di), lambda i: (0, i, 0)),
            pl.BlockSpec((RB, di), lambda i: (i, 0)),
            pl.BlockSpec((NC, RB, DW), lambda i: (0, i, 0)),
            pl.BlockSpec((1, di), lambda i: (0, 0)),
            pl.BlockSpec((di, do), lambda i: (0, 0)),
        ],
        out_specs=pl.BlockSpec((RB, do), lambda i: (i, 0)),
        out_shape=jax.ShapeDtypeStruct((NP, do), _f32),
    )(p, zp, degp, b2d, W)


def _pool_body(p_ref, zp_ref, degp_ref, b3_ref, batch_ref,
               wf1_ref, bf1_ref, wf2_ref, bf2_ref, out_ref,
               sums_ref, cnts_ref):
    i = pl.program_id(0)

    @pl.when(i == 0)
    def _init():
        sums_ref[...] = jnp.zeros_like(sums_ref)
        cnts_ref[...] = jnp.zeros_like(cnts_ref)

    dinv = _dinv_of(degp_ref[...])
    sm = p_ref[0] + p_ref[1] + zp_ref[...]
    h3 = jnp.maximum(dinv[:, None] * sm + b3_ref[...], 0.0)
    b = batch_ref[0]
    gids = lax.broadcasted_iota(jnp.int32, (G, RB), 0)
    onehot = (gids == b).astype(_f32)
    sums_ref[...] += jnp.dot(onehot, h3, preferred_element_type=_f32)
    cnts_ref[...] += jnp.sum(onehot, axis=1, keepdims=True)

    @pl.when(i == NB - 1)
    def _final():
        g = sums_ref[...] / jnp.maximum(cnts_ref[...], 1.0)
        f = jnp.maximum(jnp.dot(g, wf1_ref[...],
                                preferred_element_type=_f32) + bf1_ref[...], 0.0)
        out_ref[...] = jnp.dot(f, wf2_ref[...],
                               preferred_element_type=_f32) + bf2_ref[...]


def _tc_pool(p, zp, degp, b3_2d, batch3, Wfc1, bfc1_2d, Wfc2, bfc2_2d):
    return pl.pallas_call(
        _pool_body,
        grid=(NB,),
        in_specs=[
            pl.BlockSpec((NC, RB, 32), lambda i: (0, i, 0)),
            pl.BlockSpec((RB, 32), lambda i: (i, 0)),
            pl.BlockSpec((NC, RB, DW), lambda i: (0, i, 0)),
            pl.BlockSpec((1, 32), lambda i: (0, 0)),
            pl.BlockSpec((1, 1, RB), lambda i: (i, 0, 0)),
            pl.BlockSpec((32, 16), lambda i: (0, 0)),
            pl.BlockSpec((1, 16), lambda i: (0, 0)),
            pl.BlockSpec((16, 1), lambda i: (0, 0)),
            pl.BlockSpec((1, 1), lambda i: (0, 0)),
        ],
        out_specs=pl.BlockSpec((G, 1), lambda i: (0, 0)),
        out_shape=jax.ShapeDtypeStruct((G, 1), _f32),
        scratch_shapes=[
            pltpu.VMEM((G, 32), _f32),
            pltpu.VMEM((G, 1), _f32),
        ],
    )(p, zp, degp, b3_2d, batch3, Wfc1, bfc1_2d, Wfc2, bfc2_2d)


def kernel(x, edge_index, batch, W1, b1, W2, b2, W3, b3,
           Wfc1, bfc1, Wfc2, bfc2):
    src_flat = jnp.concatenate(
        [edge_index[0], jnp.zeros((EPAD - E,), jnp.int32)])
    dst_flat = jnp.concatenate(
        [edge_index[1], jnp.full((EPAD - E,), DUMP, jnp.int32)])
    src = src_flat.reshape(EPAD // EC, EC)
    dst = dst_flat.reshape(EPAD // EC, EC)
    src_n = src_flat.reshape(EPAD // EC128, EC128)
    dst_n = dst_flat.reshape(EPAD // EC128, EC128)
    dstw = dst_flat.reshape(NW, CH, EC)

    xp = jnp.pad(x, ((0, NP - N), (0, 0)))
    batch3 = jnp.pad(batch, (0, NP - N), constant_values=G - 1).reshape(NB, 1, RB)

    degp = _deg_kernel(dstw, jnp.ones((EC, DW), _f32), jnp.zeros((NP, DW), _f32))

    z1p = _tc_layer1(xp, W1, degp)
    p1 = _scatter128(z1p, src_n, dst_n, jnp.zeros((NP, 128), _f32))
    z2p = _tc_layer(p1, z1p, degp, b1.reshape(1, -1), W2)
    p2 = _scatter64(z2p, src, dst, jnp.zeros((NP, 64), _f32))
    z3p = _tc_layer(p2, z2p, degp, b2.reshape(1, -1), W3)
    p3 = _scatter32(z3p, src, dst, jnp.zeros((NP, 32), _f32))

    out = _tc_pool(p3, z3p, degp, b3.reshape(1, -1), batch3,
                   Wfc1, bfc1.reshape(1, -1), Wfc2, bfc2.reshape(1, -1))
    return out[:NG]
